# baseline (speedup 1.0000x reference)
import functools

import jax
import jax.numpy as jnp
from jax import lax
from jax.experimental import pallas as pl
from jax.experimental.pallas import tpu as pltpu
from jax.experimental.pallas import tpu_sc as plsc

_TOPK = 9
_NUM_CLASSES = 80
_BG = _NUM_CLASSES
_LEVELS = ((0, 6400, 0), (6400, 1600, 0), (7936, 464, 64))
_NC = 27
_F = 2

_NW = 32
_A = 8400
_B = 16
_CHUNK = (_B * _A) // _NW
_CROWS = 33
_CPAD = _CROWS * 128
_SPAN = _CHUNK * _NUM_CLASSES
_ZBUF = 8000
_NZDMA = _SPAN // _ZBUF


def _body(anch_ref, gt_ref, lab_ref, pd_ref, tl_ref, tb_ref, si_ref, sv_ref):
    A = anch_ref.shape[1]
    M = gt_ref.shape[1]
    FM = _F * M
    INF = jnp.float32(jnp.inf)
    BIG = jnp.int32(1 << 30)

    anch = anch_ref[...]
    ax1, ay1 = anch[0:1, :], anch[1:2, :]
    ax2, ay2 = anch[2:3, :], anch[3:4, :]
    gf = jnp.concatenate([gt_ref[j] for j in range(_F)], axis=0)
    gx1, gy1 = gf[:, 0:1], gf[:, 1:2]
    gx2, gy2 = gf[:, 2:3], gf[:, 3:4]

    acx = (ax1 + ax2) * 0.5
    acy = (ay1 + ay2) * 0.5
    gcx = (gx1 + gx2) * 0.5
    gcy = (gy1 + gy2) * 0.5
    dx = gcx - acx
    dy = gcy - acy
    d = dx * dx + dy * dy

    ga = (gx2 - gx1) * (gy2 - gy1)
    aa = (ax2 - ax1) * (ay2 - ay1)
    wx = jnp.clip(jnp.minimum(gx2, ax2) - jnp.maximum(gx1, ax1), 0.0)
    wy = jnp.clip(jnp.minimum(gy2, ay2) - jnp.maximum(gy1, ay1), 0.0)
    inter = wx * wy
    ov = inter / jnp.maximum(ga + aa - inter, 1e-6)

    iota_a = jax.lax.broadcasted_iota(jnp.int32, (1, A), 1)

    d9s = []
    for start, width, prefix in _LEVELS:
        dl = jax.lax.slice(d, (0, start), (FM, start + width))
        if prefix:
            il = jax.lax.broadcasted_iota(jnp.int32, (1, width), 1)
            dl = jnp.where(il >= prefix, dl, INF)

        def step(_, carry):
            dc, _ = carry
            v = jnp.min(dc, axis=1, keepdims=True)
            dc = jnp.where(dc == v, INF, dc)
            return dc, v

        _, vmin = jax.lax.fori_loop(
            0, _TOPK, step, (dl, jnp.zeros((FM, 1), jnp.float32)))
        d9s.append(vmin)

    lvl0 = iota_a < 6400
    lvl1 = (iota_a >= 6400) & (iota_a < 8000)
    d9f = jnp.where(lvl0, d9s[0], jnp.where(lvl1, d9s[1], d9s[2]))
    is_in = d <= d9f

    s1 = jnp.sum(jnp.where(is_in, ov, 0.0), axis=1, keepdims=True)
    mean = s1 * (1.0 / _NC)
    dev = ov - mean
    s2 = jnp.sum(jnp.where(is_in, dev * dev, 0.0), axis=1, keepdims=True)
    thr = mean + jnp.sqrt(s2 * (1.0 / (_NC - 1)))

    ing = (jnp.minimum(jnp.minimum(acx - gx1, acy - gy1),
                       jnp.minimum(gx2 - acx, gy2 - acy)) > 1e-9)
    mask_pos = is_in & (ov > thr) & ing

    iota_m = jax.lax.broadcasted_iota(jnp.int32, (M, 1), 0)
    dnum = (((0,), (0,)), ((), ()))

    for j in range(_F):
        mp = jax.lax.slice(mask_pos, (j * M, 0), ((j + 1) * M, A))
        ovj = jax.lax.slice(ov, (j * M, 0), ((j + 1) * M, A))
        cnt = jnp.sum(mp.astype(jnp.int32), axis=0, keepdims=True)
        first_m = jnp.min(jnp.where(mp, iota_m, BIG), axis=0, keepdims=True)
        first_m = jnp.where(cnt > 0, first_m, 0)
        best_ov = jnp.max(ovj, axis=0, keepdims=True)
        best_m = jnp.min(jnp.where(ovj == best_ov, iota_m, BIG), axis=0,
                         keepdims=True)
        mstar = jnp.where(cnt > 1, best_m, first_m)
        fgv = jnp.where(cnt > 1, 1, cnt)

        ohb = mstar == iota_m
        oh = ohb.astype(jnp.float32)
        lab = lab_ref[j]
        gj = gt_ref[j]
        bx1 = jax.lax.slice(gx1, (j * M, 0), ((j + 1) * M, 1))
        by1 = jax.lax.slice(gy1, (j * M, 0), ((j + 1) * M, 1))
        bx2 = jax.lax.slice(gx2, (j * M, 0), ((j + 1) * M, 1))
        by2 = jax.lax.slice(gy2, (j * M, 0), ((j + 1) * M, 1))
        sx1 = jnp.sum(jnp.where(ohb, bx1, 0.0), axis=0, keepdims=True)
        sy1 = jnp.sum(jnp.where(ohb, by1, 0.0), axis=0, keepdims=True)
        sx2 = jnp.sum(jnp.where(ohb, bx2, 0.0), axis=0, keepdims=True)
        sy2 = jnp.sum(jnp.where(ohb, by2, 0.0), axis=0, keepdims=True)
        slab = jnp.sum(jnp.where(ohb, lab, 0.0), axis=0, keepdims=True)

        p = pd_ref[j]
        px1, py1, px2, py2 = p[0:1, :], p[1:2, :], p[2:3, :], p[3:4, :]
        ox = jnp.clip(jnp.maximum(sx1, px1) - jnp.minimum(sx2, px2), 0.0)
        oy = jnp.clip(jnp.maximum(sy1, py1) - jnp.minimum(sy2, py2), 0.0)
        inter2 = ox * oy
        pa = jnp.clip(sx2 - sx1, 0.0) * jnp.clip(sy2 - sy1, 0.0)
        qa = jnp.clip(px2 - px1, 0.0) * jnp.clip(py2 - py1, 0.0)
        iou_pd = inter2 / (pa + qa - inter2 + 1e-9)
        val = jnp.where(fgv > 0, jnp.maximum(iou_pd, 0.0), 0.0)

        tl = jnp.where(fgv > 0, slab.astype(jnp.int32), _BG)
        tl_ref[j] = tl

        tb_ref[j] = jax.lax.dot_general(oh, gj, dnum,
                                        preferred_element_type=jnp.float32)

        bglob = pl.program_id(0) * _F + j
        si_ref[j] = (bglob * A + iota_a) * _NUM_CLASSES \
            + jnp.minimum(tl, _NUM_CLASSES - 1)
        sv_ref[j] = val


def _sc_scatter_body(idx_hbm, val_hbm, out_hbm, idx_v, val_v, zbuf,
                     zsem, ssem):
    wid = lax.axis_index("s") * 2 + lax.axis_index("c")
    base = wid * _SPAN

    def zstep(i, _):
        zbuf[pl.ds(i * 16, 16)] = jnp.zeros((16,), jnp.float32)
        return 0

    lax.fori_loop(0, _ZBUF // 16, zstep, 0)
    zcopies = [
        pltpu.async_copy(zbuf, out_hbm.at[pl.ds(base + k * _ZBUF, _ZBUF)],
                         zsem)
        for k in range(_NZDMA)
    ]
    pltpu.sync_copy(idx_hbm.at[wid], idx_v)
    pltpu.sync_copy(val_hbm.at[wid], val_v)
    for c in zcopies:
        c.wait()
    scopies = [
        pltpu.async_copy(val_v.at[r], out_hbm.at[idx_v.at[r]], ssem)
        for r in range(_CROWS)
    ]
    for c in scopies:
        c.wait()


def _sc_scatter(idx_pad, val_pad):
    mesh = plsc.VectorSubcoreMesh(core_axis_name="c", subcore_axis_name="s")
    k = functools.partial(
        pl.kernel,
        mesh=mesh,
        out_type=jax.ShapeDtypeStruct((_B * _A * _NUM_CLASSES,), jnp.float32),
        scratch_types=[
            pltpu.VMEM((_CROWS, 128), jnp.int32),
            pltpu.VMEM((_CROWS, 128), jnp.float32),
            pltpu.VMEM((_ZBUF,), jnp.float32),
            pltpu.SemaphoreType.DMA,
            pltpu.SemaphoreType.DMA,
        ],
    )(_sc_scatter_body)
    return k(idx_pad, val_pad)


def kernel(anchor_bboxes, n_level_bboxes, gt_labels, gt_bboxes, mask_gt,
           pd_bboxes):
    A = anchor_bboxes.shape[0]
    B, M, _ = gt_bboxes.shape
    anchors_t = anchor_bboxes.T
    pd_t = jnp.transpose(pd_bboxes, (0, 2, 1))
    lab = gt_labels.astype(jnp.float32)

    tl3, tb, si3, sv3 = pl.pallas_call(
        _body,
        grid=(B // _F,),
        in_specs=[
            pl.BlockSpec((4, A), lambda b: (0, 0)),
            pl.BlockSpec((_F, M, 4), lambda b: (b, 0, 0)),
            pl.BlockSpec((_F, M, 1), lambda b: (b, 0, 0)),
            pl.BlockSpec((_F, 4, A), lambda b: (b, 0, 0)),
        ],
        out_specs=[
            pl.BlockSpec((_F, 1, A), lambda b: (b, 0, 0)),
            pl.BlockSpec((_F, A, 4), lambda b: (b, 0, 0)),
            pl.BlockSpec((_F, 1, A), lambda b: (b, 0, 0)),
            pl.BlockSpec((_F, 1, A), lambda b: (b, 0, 0)),
        ],
        out_shape=[
            jax.ShapeDtypeStruct((B, 1, A), jnp.int32),
            jax.ShapeDtypeStruct((B, A, 4), jnp.float32),
            jax.ShapeDtypeStruct((B, 1, A), jnp.int32),
            jax.ShapeDtypeStruct((B, 1, A), jnp.float32),
        ],
        compiler_params=pltpu.CompilerParams(
            dimension_semantics=("arbitrary",)),
    )(anchors_t, gt_bboxes, lab, pd_t)

    tl = tl3[:, 0, :]
    fg = tl != _BG

    idx_w = si3.reshape(_NW, _CHUNK)
    val_w = sv3.reshape(_NW, _CHUNK)
    idx_pad = jnp.pad(idx_w, ((0, 0), (0, _CPAD - _CHUNK)), mode="edge")
    val_pad = jnp.pad(val_w, ((0, 0), (0, _CPAD - _CHUNK)), mode="edge")
    ts = _sc_scatter(idx_pad.reshape(_NW, _CROWS, 128),
                     val_pad.reshape(_NW, _CROWS, 128))
    ts = ts.reshape(B, A, _NUM_CLASSES)
    return tl, tb, ts, fg


# speedup vs baseline: 1.0072x; 1.0072x over previous
import functools

import jax
import jax.numpy as jnp
from jax import lax
from jax.experimental import pallas as pl
from jax.experimental.pallas import tpu as pltpu
from jax.experimental.pallas import tpu_sc as plsc

_TOPK = 9
_NUM_CLASSES = 80
_BG = _NUM_CLASSES
_LEVELS = ((0, 6400, 0), (6400, 1600, 0), (7936, 464, 64))
_NC = 27
_F = 2

_NW = 32
_A = 8400
_B = 16
_ROWS = (_B * _A) // 128
_HI = _ROWS - 32 * _NW
_RPP = 128 * _NUM_CLASSES
_ZBUF = 107520


def _body(anch_ref, gt_ref, lab_ref, pd_ref, tl_ref, tb_ref, si_ref, sv_ref):
    A = anch_ref.shape[1]
    M = gt_ref.shape[1]
    FM = _F * M
    INF = jnp.float32(jnp.inf)
    BIG = jnp.int32(1 << 30)

    anch = anch_ref[...]
    ax1, ay1 = anch[0:1, :], anch[1:2, :]
    ax2, ay2 = anch[2:3, :], anch[3:4, :]
    gf = jnp.concatenate([gt_ref[j] for j in range(_F)], axis=0)
    gx1, gy1 = gf[:, 0:1], gf[:, 1:2]
    gx2, gy2 = gf[:, 2:3], gf[:, 3:4]

    acx = (ax1 + ax2) * 0.5
    acy = (ay1 + ay2) * 0.5
    gcx = (gx1 + gx2) * 0.5
    gcy = (gy1 + gy2) * 0.5
    dx = gcx - acx
    dy = gcy - acy
    d = dx * dx + dy * dy

    ga = (gx2 - gx1) * (gy2 - gy1)
    aa = (ax2 - ax1) * (ay2 - ay1)
    wx = jnp.clip(jnp.minimum(gx2, ax2) - jnp.maximum(gx1, ax1), 0.0)
    wy = jnp.clip(jnp.minimum(gy2, ay2) - jnp.maximum(gy1, ay1), 0.0)
    inter = wx * wy
    ov = inter / jnp.maximum(ga + aa - inter, 1e-6)

    iota_a = jax.lax.broadcasted_iota(jnp.int32, (1, A), 1)

    d9s = []
    for start, width, prefix in _LEVELS:
        dl = jax.lax.slice(d, (0, start), (FM, start + width))
        if prefix:
            il = jax.lax.broadcasted_iota(jnp.int32, (1, width), 1)
            dl = jnp.where(il >= prefix, dl, INF)

        def step(_, carry):
            dc, _ = carry
            v = jnp.min(dc, axis=1, keepdims=True)
            dc = jnp.where(dc == v, INF, dc)
            return dc, v

        _, vmin = jax.lax.fori_loop(
            0, _TOPK, step, (dl, jnp.zeros((FM, 1), jnp.float32)))
        d9s.append(vmin)

    lvl0 = iota_a < 6400
    lvl1 = (iota_a >= 6400) & (iota_a < 8000)
    d9f = jnp.where(lvl0, d9s[0], jnp.where(lvl1, d9s[1], d9s[2]))
    is_in = d <= d9f

    s1 = jnp.sum(jnp.where(is_in, ov, 0.0), axis=1, keepdims=True)
    mean = s1 * (1.0 / _NC)
    dev = ov - mean
    s2 = jnp.sum(jnp.where(is_in, dev * dev, 0.0), axis=1, keepdims=True)
    thr = mean + jnp.sqrt(s2 * (1.0 / (_NC - 1)))

    ing = (jnp.minimum(jnp.minimum(acx - gx1, acy - gy1),
                       jnp.minimum(gx2 - acx, gy2 - acy)) > 1e-9)
    mask_pos = is_in & (ov > thr) & ing

    iota_m = jax.lax.broadcasted_iota(jnp.int32, (M, 1), 0)
    dnum = (((0,), (0,)), ((), ()))

    for j in range(_F):
        mp = jax.lax.slice(mask_pos, (j * M, 0), ((j + 1) * M, A))
        ovj = jax.lax.slice(ov, (j * M, 0), ((j + 1) * M, A))
        cnt = jnp.sum(mp.astype(jnp.int32), axis=0, keepdims=True)
        first_m = jnp.min(jnp.where(mp, iota_m, BIG), axis=0, keepdims=True)
        first_m = jnp.where(cnt > 0, first_m, 0)
        best_ov = jnp.max(ovj, axis=0, keepdims=True)
        best_m = jnp.min(jnp.where(ovj == best_ov, iota_m, BIG), axis=0,
                         keepdims=True)
        mstar = jnp.where(cnt > 1, best_m, first_m)
        fgv = jnp.where(cnt > 1, 1, cnt)

        ohb = mstar == iota_m
        oh = ohb.astype(jnp.float32)
        lab = lab_ref[j]
        gj = gt_ref[j]
        bx1 = jax.lax.slice(gx1, (j * M, 0), ((j + 1) * M, 1))
        by1 = jax.lax.slice(gy1, (j * M, 0), ((j + 1) * M, 1))
        bx2 = jax.lax.slice(gx2, (j * M, 0), ((j + 1) * M, 1))
        by2 = jax.lax.slice(gy2, (j * M, 0), ((j + 1) * M, 1))
        sx1 = jnp.sum(jnp.where(ohb, bx1, 0.0), axis=0, keepdims=True)
        sy1 = jnp.sum(jnp.where(ohb, by1, 0.0), axis=0, keepdims=True)
        sx2 = jnp.sum(jnp.where(ohb, bx2, 0.0), axis=0, keepdims=True)
        sy2 = jnp.sum(jnp.where(ohb, by2, 0.0), axis=0, keepdims=True)
        slab = jnp.sum(jnp.where(ohb, lab, 0.0), axis=0, keepdims=True)

        p = pd_ref[j]
        px1, py1, px2, py2 = p[0:1, :], p[1:2, :], p[2:3, :], p[3:4, :]
        ox = jnp.clip(jnp.maximum(sx1, px1) - jnp.minimum(sx2, px2), 0.0)
        oy = jnp.clip(jnp.maximum(sy1, py1) - jnp.minimum(sy2, py2), 0.0)
        inter2 = ox * oy
        pa = jnp.clip(sx2 - sx1, 0.0) * jnp.clip(sy2 - sy1, 0.0)
        qa = jnp.clip(px2 - px1, 0.0) * jnp.clip(py2 - py1, 0.0)
        iou_pd = inter2 / (pa + qa - inter2 + 1e-9)
        val = jnp.where(fgv > 0, jnp.maximum(iou_pd, 0.0), 0.0)

        tl = jnp.where(fgv > 0, slab.astype(jnp.int32), _BG)
        tl_ref[j] = tl

        tb_ref[j] = jax.lax.dot_general(oh, gj, dnum,
                                        preferred_element_type=jnp.float32)

        bglob = pl.program_id(0) * _F + j
        si_ref[j] = (bglob * A + iota_a) * _NUM_CLASSES \
            + jnp.minimum(tl, _NUM_CLASSES - 1)
        sv_ref[j] = val


def _sc_scatter_body(idx_hbm, val_hbm, out_hbm, iv33, vv33, iv32, vv32,
                     zbuf, zsem, ssem):
    wid = lax.axis_index("s") * 2 + lax.axis_index("c")

    def zstep(i, _):
        b = i * 128
        for u in range(8):
            zbuf[pl.ds(b + u * 16, 16)] = jnp.zeros((16,), jnp.float32)
        return 0

    lax.fori_loop(0, _ZBUF // 128, zstep, 0)

    def run(nrows, iv, vv):
        if nrows == 33:
            row0 = wid * 33
        else:
            row0 = _HI * 33 + (wid - _HI) * 32
        span = nrows * _RPP
        base = row0 * _RPP
        n = nrows * 128
        zcs = []
        off = 0
        while off < span:
            sz = min(_ZBUF, span - off)
            zcs.append(pltpu.async_copy(
                zbuf.at[pl.ds(0, sz)], out_hbm.at[pl.ds(base + off, sz)],
                zsem))
            off += sz
        pltpu.sync_copy(idx_hbm.at[pl.ds(row0 * 128, n)], iv)
        pltpu.sync_copy(val_hbm.at[pl.ds(row0 * 128, n)], vv)
        for c in zcs:
            c.wait()
        pltpu.async_copy(vv, out_hbm.at[iv], ssem).wait()

    @pl.when(wid < _HI)
    def _():
        run(33, iv33, vv33)

    @pl.when(wid >= _HI)
    def _():
        run(32, iv32, vv32)


def _sc_scatter(idx_flat, val_flat):
    mesh = plsc.VectorSubcoreMesh(core_axis_name="c", subcore_axis_name="s")
    k = functools.partial(
        pl.kernel,
        mesh=mesh,
        out_type=jax.ShapeDtypeStruct((_B * _A * _NUM_CLASSES,), jnp.float32),
        scratch_types=[
            pltpu.VMEM((33 * 128,), jnp.int32),
            pltpu.VMEM((33 * 128,), jnp.float32),
            pltpu.VMEM((32 * 128,), jnp.int32),
            pltpu.VMEM((32 * 128,), jnp.float32),
            pltpu.VMEM((_ZBUF,), jnp.float32),
            pltpu.SemaphoreType.DMA,
            pltpu.SemaphoreType.DMA,
        ],
    )(_sc_scatter_body)
    return k(idx_flat, val_flat)


def kernel(anchor_bboxes, n_level_bboxes, gt_labels, gt_bboxes, mask_gt,
           pd_bboxes):
    A = anchor_bboxes.shape[0]
    B, M, _ = gt_bboxes.shape
    anchors_t = anchor_bboxes.T
    pd_t = jnp.transpose(pd_bboxes, (0, 2, 1))
    lab = gt_labels.astype(jnp.float32)

    tl3, tb, si3, sv3 = pl.pallas_call(
        _body,
        grid=(B // _F,),
        in_specs=[
            pl.BlockSpec((4, A), lambda b: (0, 0)),
            pl.BlockSpec((_F, M, 4), lambda b: (b, 0, 0)),
            pl.BlockSpec((_F, M, 1), lambda b: (b, 0, 0)),
            pl.BlockSpec((_F, 4, A), lambda b: (b, 0, 0)),
        ],
        out_specs=[
            pl.BlockSpec((_F, 1, A), lambda b: (b, 0, 0)),
            pl.BlockSpec((_F, A, 4), lambda b: (b, 0, 0)),
            pl.BlockSpec((_F, 1, A), lambda b: (b, 0, 0)),
            pl.BlockSpec((_F, 1, A), lambda b: (b, 0, 0)),
        ],
        out_shape=[
            jax.ShapeDtypeStruct((B, 1, A), jnp.int32),
            jax.ShapeDtypeStruct((B, A, 4), jnp.float32),
            jax.ShapeDtypeStruct((B, 1, A), jnp.int32),
            jax.ShapeDtypeStruct((B, 1, A), jnp.float32),
        ],
        compiler_params=pltpu.CompilerParams(
            dimension_semantics=("arbitrary",)),
    )(anchors_t, gt_bboxes, lab, pd_t)

    tl = tl3[:, 0, :]
    fg = tl != _BG

    ts = _sc_scatter(si3.reshape(_B * _A), sv3.reshape(_B * _A))
    ts = ts.reshape(B, A, _NUM_CLASSES)
    return tl, tb, ts, fg


# speedup vs baseline: 1.1091x; 1.1011x over previous
import functools

import jax
import jax.numpy as jnp
from jax import lax
from jax.experimental import pallas as pl
from jax.experimental.pallas import tpu as pltpu
from jax.experimental.pallas import tpu_sc as plsc

_TOPK = 9
_NUM_CLASSES = 80
_BG = _NUM_CLASSES
_LEVELS = ((0, 6400, 0), (6400, 1600, 0), (7936, 464, 64))
_NC = 27
_F = 2

_NW = 32
_A = 8400
_B = 16
_ROWS = (_B * _A) // 128
_HI = _ROWS - 32 * _NW
_RPP = 128 * _NUM_CLASSES
_ZBUF = 107520


def _body(anch_ref, gt_ref, lab_ref, pd_ref, tl_ref, tb_ref, si_ref, sv_ref):
    A = anch_ref.shape[1]
    M = gt_ref.shape[1]
    FM = _F * M
    INF = jnp.float32(jnp.inf)
    BIG = jnp.int32(1 << 30)

    anch = anch_ref[...]
    ax1, ay1 = anch[0:1, :], anch[1:2, :]
    ax2, ay2 = anch[2:3, :], anch[3:4, :]
    gf = jnp.concatenate([gt_ref[j] for j in range(_F)], axis=0)
    gx1, gy1 = gf[:, 0:1], gf[:, 1:2]
    gx2, gy2 = gf[:, 2:3], gf[:, 3:4]

    acx = (ax1 + ax2) * 0.5
    acy = (ay1 + ay2) * 0.5
    gcx = (gx1 + gx2) * 0.5
    gcy = (gy1 + gy2) * 0.5
    dx = gcx - acx
    dy = gcy - acy
    d = dx * dx + dy * dy

    ga = (gx2 - gx1) * (gy2 - gy1)
    aa = (ax2 - ax1) * (ay2 - ay1)
    wx = jnp.clip(jnp.minimum(gx2, ax2) - jnp.maximum(gx1, ax1), 0.0)
    wy = jnp.clip(jnp.minimum(gy2, ay2) - jnp.maximum(gy1, ay1), 0.0)
    inter = wx * wy
    ov = inter / jnp.maximum(ga + aa - inter, 1e-6)

    iota_a = jax.lax.broadcasted_iota(jnp.int32, (1, A), 1)

    d9s = []
    for start, width, prefix in _LEVELS:
        dl = jax.lax.slice(d, (0, start), (FM, start + width))
        if prefix:
            il = jax.lax.broadcasted_iota(jnp.int32, (1, width), 1)
            dl = jnp.where(il >= prefix, dl, INF)

        dc = dl
        vmin = None
        for _ in range(_TOPK):
            vmin = jnp.min(dc, axis=1, keepdims=True)
            dc = jnp.where(dc == vmin, INF, dc)
        d9s.append(vmin)

    lvl0 = iota_a < 6400
    lvl1 = (iota_a >= 6400) & (iota_a < 8000)
    d9f = jnp.where(lvl0, d9s[0], jnp.where(lvl1, d9s[1], d9s[2]))
    is_in = d <= d9f

    s1 = jnp.sum(jnp.where(is_in, ov, 0.0), axis=1, keepdims=True)
    mean = s1 * (1.0 / _NC)
    dev = ov - mean
    s2 = jnp.sum(jnp.where(is_in, dev * dev, 0.0), axis=1, keepdims=True)
    thr = mean + jnp.sqrt(s2 * (1.0 / (_NC - 1)))

    ing = (jnp.minimum(jnp.minimum(acx - gx1, acy - gy1),
                       jnp.minimum(gx2 - acx, gy2 - acy)) > 1e-9)
    mask_pos = is_in & (ov > thr) & ing

    iota_m = jax.lax.broadcasted_iota(jnp.int32, (M, 1), 0)
    dnum = (((0,), (0,)), ((), ()))

    for j in range(_F):
        mp = jax.lax.slice(mask_pos, (j * M, 0), ((j + 1) * M, A))
        ovj = jax.lax.slice(ov, (j * M, 0), ((j + 1) * M, A))
        cnt = jnp.sum(mp.astype(jnp.int32), axis=0, keepdims=True)
        first_m = jnp.min(jnp.where(mp, iota_m, BIG), axis=0, keepdims=True)
        first_m = jnp.where(cnt > 0, first_m, 0)
        best_ov = jnp.max(ovj, axis=0, keepdims=True)
        best_m = jnp.min(jnp.where(ovj == best_ov, iota_m, BIG), axis=0,
                         keepdims=True)
        mstar = jnp.where(cnt > 1, best_m, first_m)
        fgv = jnp.where(cnt > 1, 1, cnt)

        ohb = mstar == iota_m
        oh = ohb.astype(jnp.float32)
        lab = lab_ref[j]
        gj = gt_ref[j]
        bx1 = jax.lax.slice(gx1, (j * M, 0), ((j + 1) * M, 1))
        by1 = jax.lax.slice(gy1, (j * M, 0), ((j + 1) * M, 1))
        bx2 = jax.lax.slice(gx2, (j * M, 0), ((j + 1) * M, 1))
        by2 = jax.lax.slice(gy2, (j * M, 0), ((j + 1) * M, 1))
        sx1 = jnp.sum(jnp.where(ohb, bx1, 0.0), axis=0, keepdims=True)
        sy1 = jnp.sum(jnp.where(ohb, by1, 0.0), axis=0, keepdims=True)
        sx2 = jnp.sum(jnp.where(ohb, bx2, 0.0), axis=0, keepdims=True)
        sy2 = jnp.sum(jnp.where(ohb, by2, 0.0), axis=0, keepdims=True)
        slab = jnp.sum(jnp.where(ohb, lab, 0.0), axis=0, keepdims=True)

        p = pd_ref[j]
        px1, py1, px2, py2 = p[0:1, :], p[1:2, :], p[2:3, :], p[3:4, :]
        ox = jnp.clip(jnp.maximum(sx1, px1) - jnp.minimum(sx2, px2), 0.0)
        oy = jnp.clip(jnp.maximum(sy1, py1) - jnp.minimum(sy2, py2), 0.0)
        inter2 = ox * oy
        pa = jnp.clip(sx2 - sx1, 0.0) * jnp.clip(sy2 - sy1, 0.0)
        qa = jnp.clip(px2 - px1, 0.0) * jnp.clip(py2 - py1, 0.0)
        iou_pd = inter2 / (pa + qa - inter2 + 1e-9)
        val = jnp.where(fgv > 0, jnp.maximum(iou_pd, 0.0), 0.0)

        tl = jnp.where(fgv > 0, slab.astype(jnp.int32), _BG)
        tl_ref[j] = tl

        tb_ref[j] = jax.lax.dot_general(oh, gj, dnum,
                                        preferred_element_type=jnp.float32)

        bglob = pl.program_id(0) * _F + j
        si_ref[j] = (bglob * A + iota_a) * _NUM_CLASSES \
            + jnp.minimum(tl, _NUM_CLASSES - 1)
        sv_ref[j] = val


def _sc_scatter_body(idx_hbm, val_hbm, out_hbm, iv33, vv33, iv32, vv32,
                     zbuf, zsem, ssem):
    wid = lax.axis_index("s") * 2 + lax.axis_index("c")

    def zstep(i, _):
        b = i * 128
        for u in range(8):
            zbuf[pl.ds(b + u * 16, 16)] = jnp.zeros((16,), jnp.float32)
        return 0

    lax.fori_loop(0, _ZBUF // 128, zstep, 0)

    def run(nrows, iv, vv):
        if nrows == 33:
            row0 = wid * 33
        else:
            row0 = _HI * 33 + (wid - _HI) * 32
        span = nrows * _RPP
        base = row0 * _RPP
        n = nrows * 128
        zcs = []
        off = 0
        while off < span:
            sz = min(_ZBUF, span - off)
            zcs.append(pltpu.async_copy(
                zbuf.at[pl.ds(0, sz)], out_hbm.at[pl.ds(base + off, sz)],
                zsem))
            off += sz
        pltpu.sync_copy(idx_hbm.at[pl.ds(row0 * 128, n)], iv)
        pltpu.sync_copy(val_hbm.at[pl.ds(row0 * 128, n)], vv)
        for c in zcs:
            c.wait()
        pltpu.async_copy(vv, out_hbm.at[iv], ssem).wait()

    @pl.when(wid < _HI)
    def _():
        run(33, iv33, vv33)

    @pl.when(wid >= _HI)
    def _():
        run(32, iv32, vv32)


def _sc_scatter(idx_flat, val_flat):
    mesh = plsc.VectorSubcoreMesh(core_axis_name="c", subcore_axis_name="s")
    k = functools.partial(
        pl.kernel,
        mesh=mesh,
        out_type=jax.ShapeDtypeStruct((_B * _A * _NUM_CLASSES,), jnp.float32),
        scratch_types=[
            pltpu.VMEM((33 * 128,), jnp.int32),
            pltpu.VMEM((33 * 128,), jnp.float32),
            pltpu.VMEM((32 * 128,), jnp.int32),
            pltpu.VMEM((32 * 128,), jnp.float32),
            pltpu.VMEM((_ZBUF,), jnp.float32),
            pltpu.SemaphoreType.DMA,
            pltpu.SemaphoreType.DMA,
        ],
    )(_sc_scatter_body)
    return k(idx_flat, val_flat)


def kernel(anchor_bboxes, n_level_bboxes, gt_labels, gt_bboxes, mask_gt,
           pd_bboxes):
    A = anchor_bboxes.shape[0]
    B, M, _ = gt_bboxes.shape
    anchors_t = anchor_bboxes.T
    pd_t = jnp.transpose(pd_bboxes, (0, 2, 1))
    lab = gt_labels.astype(jnp.float32)

    tl3, tb, si3, sv3 = pl.pallas_call(
        _body,
        grid=(B // _F,),
        in_specs=[
            pl.BlockSpec((4, A), lambda b: (0, 0)),
            pl.BlockSpec((_F, M, 4), lambda b: (b, 0, 0)),
            pl.BlockSpec((_F, M, 1), lambda b: (b, 0, 0)),
            pl.BlockSpec((_F, 4, A), lambda b: (b, 0, 0)),
        ],
        out_specs=[
            pl.BlockSpec((_F, 1, A), lambda b: (b, 0, 0)),
            pl.BlockSpec((_F, A, 4), lambda b: (b, 0, 0)),
            pl.BlockSpec((_F, 1, A), lambda b: (b, 0, 0)),
            pl.BlockSpec((_F, 1, A), lambda b: (b, 0, 0)),
        ],
        out_shape=[
            jax.ShapeDtypeStruct((B, 1, A), jnp.int32),
            jax.ShapeDtypeStruct((B, A, 4), jnp.float32),
            jax.ShapeDtypeStruct((B, 1, A), jnp.int32),
            jax.ShapeDtypeStruct((B, 1, A), jnp.float32),
        ],
        compiler_params=pltpu.CompilerParams(
            dimension_semantics=("arbitrary",)),
    )(anchors_t, gt_bboxes, lab, pd_t)

    tl = tl3[:, 0, :]
    fg = tl != _BG

    ts = _sc_scatter(si3.reshape(_B * _A), sv3.reshape(_B * _A))
    ts = ts.reshape(B, A, _NUM_CLASSES)
    return tl, tb, ts, fg


# speedup vs baseline: 1.1093x; 1.0002x over previous
import functools

import jax
import jax.numpy as jnp
from jax import lax
from jax.experimental import pallas as pl
from jax.experimental.pallas import tpu as pltpu
from jax.experimental.pallas import tpu_sc as plsc

_TOPK = 9
_NUM_CLASSES = 80
_BG = _NUM_CLASSES
_LEVELS = ((0, 6400, 0), (6400, 1600, 0), (7936, 464, 64))
_NC = 27
_F = 4

_NW = 32
_A = 8400
_B = 16
_ROWS = (_B * _A) // 128
_HI = _ROWS - 32 * _NW
_RPP = 128 * _NUM_CLASSES
_ZBUF = 107520


def _body(anch_ref, gt_ref, lab_ref, pd_ref, tl_ref, tb_ref, si_ref, sv_ref):
    A = anch_ref.shape[1]
    M = gt_ref.shape[1]
    FM = _F * M
    INF = jnp.float32(jnp.inf)
    BIG = jnp.int32(1 << 30)

    anch = anch_ref[...]
    ax1, ay1 = anch[0:1, :], anch[1:2, :]
    ax2, ay2 = anch[2:3, :], anch[3:4, :]
    gf = jnp.concatenate([gt_ref[j] for j in range(_F)], axis=0)
    gx1, gy1 = gf[:, 0:1], gf[:, 1:2]
    gx2, gy2 = gf[:, 2:3], gf[:, 3:4]

    acx = (ax1 + ax2) * 0.5
    acy = (ay1 + ay2) * 0.5
    gcx = (gx1 + gx2) * 0.5
    gcy = (gy1 + gy2) * 0.5
    dx = gcx - acx
    dy = gcy - acy
    d = dx * dx + dy * dy

    ga = (gx2 - gx1) * (gy2 - gy1)
    aa = (ax2 - ax1) * (ay2 - ay1)
    wx = jnp.clip(jnp.minimum(gx2, ax2) - jnp.maximum(gx1, ax1), 0.0)
    wy = jnp.clip(jnp.minimum(gy2, ay2) - jnp.maximum(gy1, ay1), 0.0)
    inter = wx * wy
    ov = inter / jnp.maximum(ga + aa - inter, 1e-6)

    iota_a = jax.lax.broadcasted_iota(jnp.int32, (1, A), 1)

    d9s = []
    for start, width, prefix in _LEVELS:
        dl = jax.lax.slice(d, (0, start), (FM, start + width))
        if prefix:
            il = jax.lax.broadcasted_iota(jnp.int32, (1, width), 1)
            dl = jnp.where(il >= prefix, dl, INF)

        dc = dl
        vmin = None
        for _ in range(_TOPK):
            vmin = jnp.min(dc, axis=1, keepdims=True)
            dc = jnp.where(dc == vmin, INF, dc)
        d9s.append(vmin)

    lvl0 = iota_a < 6400
    lvl1 = (iota_a >= 6400) & (iota_a < 8000)
    d9f = jnp.where(lvl0, d9s[0], jnp.where(lvl1, d9s[1], d9s[2]))
    is_in = d <= d9f

    s1 = jnp.sum(jnp.where(is_in, ov, 0.0), axis=1, keepdims=True)
    mean = s1 * (1.0 / _NC)
    dev = ov - mean
    s2 = jnp.sum(jnp.where(is_in, dev * dev, 0.0), axis=1, keepdims=True)
    thr = mean + jnp.sqrt(s2 * (1.0 / (_NC - 1)))

    ing = (jnp.minimum(jnp.minimum(acx - gx1, acy - gy1),
                       jnp.minimum(gx2 - acx, gy2 - acy)) > 1e-9)
    mask_pos = is_in & (ov > thr) & ing

    iota_m = jax.lax.broadcasted_iota(jnp.int32, (M, 1), 0)
    dnum = (((0,), (0,)), ((), ()))

    for j in range(_F):
        mp = jax.lax.slice(mask_pos, (j * M, 0), ((j + 1) * M, A))
        ovj = jax.lax.slice(ov, (j * M, 0), ((j + 1) * M, A))
        cnt = jnp.sum(mp.astype(jnp.int32), axis=0, keepdims=True)
        first_m = jnp.min(jnp.where(mp, iota_m, BIG), axis=0, keepdims=True)
        first_m = jnp.where(cnt > 0, first_m, 0)
        best_ov = jnp.max(ovj, axis=0, keepdims=True)
        best_m = jnp.min(jnp.where(ovj == best_ov, iota_m, BIG), axis=0,
                         keepdims=True)
        mstar = jnp.where(cnt > 1, best_m, first_m)
        fgv = jnp.where(cnt > 1, 1, cnt)

        ohb = mstar == iota_m
        oh = ohb.astype(jnp.float32)
        lab = lab_ref[j]
        gj = gt_ref[j]
        bx1 = jax.lax.slice(gx1, (j * M, 0), ((j + 1) * M, 1))
        by1 = jax.lax.slice(gy1, (j * M, 0), ((j + 1) * M, 1))
        bx2 = jax.lax.slice(gx2, (j * M, 0), ((j + 1) * M, 1))
        by2 = jax.lax.slice(gy2, (j * M, 0), ((j + 1) * M, 1))
        sx1 = jnp.sum(jnp.where(ohb, bx1, 0.0), axis=0, keepdims=True)
        sy1 = jnp.sum(jnp.where(ohb, by1, 0.0), axis=0, keepdims=True)
        sx2 = jnp.sum(jnp.where(ohb, bx2, 0.0), axis=0, keepdims=True)
        sy2 = jnp.sum(jnp.where(ohb, by2, 0.0), axis=0, keepdims=True)
        slab = jnp.sum(jnp.where(ohb, lab, 0.0), axis=0, keepdims=True)

        p = pd_ref[j]
        px1, py1, px2, py2 = p[0:1, :], p[1:2, :], p[2:3, :], p[3:4, :]
        ox = jnp.clip(jnp.maximum(sx1, px1) - jnp.minimum(sx2, px2), 0.0)
        oy = jnp.clip(jnp.maximum(sy1, py1) - jnp.minimum(sy2, py2), 0.0)
        inter2 = ox * oy
        pa = jnp.clip(sx2 - sx1, 0.0) * jnp.clip(sy2 - sy1, 0.0)
        qa = jnp.clip(px2 - px1, 0.0) * jnp.clip(py2 - py1, 0.0)
        iou_pd = inter2 / (pa + qa - inter2 + 1e-9)
        val = jnp.where(fgv > 0, jnp.maximum(iou_pd, 0.0), 0.0)

        tl = jnp.where(fgv > 0, slab.astype(jnp.int32), _BG)
        tl_ref[j] = tl

        tb_ref[j] = jax.lax.dot_general(oh, gj, dnum,
                                        preferred_element_type=jnp.float32)

        bglob = pl.program_id(0) * _F + j
        si_ref[j] = (bglob * A + iota_a) * _NUM_CLASSES \
            + jnp.minimum(tl, _NUM_CLASSES - 1)
        sv_ref[j] = val


def _sc_scatter_body(idx_hbm, val_hbm, out_hbm, iv33, vv33, iv32, vv32,
                     zbuf, zsem, ssem):
    wid = lax.axis_index("s") * 2 + lax.axis_index("c")

    def zstep(i, _):
        b = i * 128
        for u in range(8):
            zbuf[pl.ds(b + u * 16, 16)] = jnp.zeros((16,), jnp.float32)
        return 0

    lax.fori_loop(0, _ZBUF // 128, zstep, 0)

    def run(nrows, iv, vv):
        if nrows == 33:
            row0 = wid * 33
        else:
            row0 = _HI * 33 + (wid - _HI) * 32
        span = nrows * _RPP
        base = row0 * _RPP
        n = nrows * 128
        zcs = []
        off = 0
        while off < span:
            sz = min(_ZBUF, span - off)
            zcs.append(pltpu.async_copy(
                zbuf.at[pl.ds(0, sz)], out_hbm.at[pl.ds(base + off, sz)],
                zsem))
            off += sz
        pltpu.sync_copy(idx_hbm.at[pl.ds(row0 * 128, n)], iv)
        pltpu.sync_copy(val_hbm.at[pl.ds(row0 * 128, n)], vv)
        for c in zcs:
            c.wait()
        pltpu.async_copy(vv, out_hbm.at[iv], ssem).wait()

    @pl.when(wid < _HI)
    def _():
        run(33, iv33, vv33)

    @pl.when(wid >= _HI)
    def _():
        run(32, iv32, vv32)


def _sc_scatter(idx_flat, val_flat):
    mesh = plsc.VectorSubcoreMesh(core_axis_name="c", subcore_axis_name="s")
    k = functools.partial(
        pl.kernel,
        mesh=mesh,
        out_type=jax.ShapeDtypeStruct((_B * _A * _NUM_CLASSES,), jnp.float32),
        scratch_types=[
            pltpu.VMEM((33 * 128,), jnp.int32),
            pltpu.VMEM((33 * 128,), jnp.float32),
            pltpu.VMEM((32 * 128,), jnp.int32),
            pltpu.VMEM((32 * 128,), jnp.float32),
            pltpu.VMEM((_ZBUF,), jnp.float32),
            pltpu.SemaphoreType.DMA,
            pltpu.SemaphoreType.DMA,
        ],
    )(_sc_scatter_body)
    return k(idx_flat, val_flat)


def kernel(anchor_bboxes, n_level_bboxes, gt_labels, gt_bboxes, mask_gt,
           pd_bboxes):
    A = anchor_bboxes.shape[0]
    B, M, _ = gt_bboxes.shape
    anchors_t = anchor_bboxes.T
    pd_t = jnp.transpose(pd_bboxes, (0, 2, 1))
    lab = gt_labels.astype(jnp.float32)

    tl3, tb, si3, sv3 = pl.pallas_call(
        _body,
        grid=(B // _F,),
        in_specs=[
            pl.BlockSpec((4, A), lambda b: (0, 0)),
            pl.BlockSpec((_F, M, 4), lambda b: (b, 0, 0)),
            pl.BlockSpec((_F, M, 1), lambda b: (b, 0, 0)),
            pl.BlockSpec((_F, 4, A), lambda b: (b, 0, 0)),
        ],
        out_specs=[
            pl.BlockSpec((_F, 1, A), lambda b: (b, 0, 0)),
            pl.BlockSpec((_F, A, 4), lambda b: (b, 0, 0)),
            pl.BlockSpec((_F, 1, A), lambda b: (b, 0, 0)),
            pl.BlockSpec((_F, 1, A), lambda b: (b, 0, 0)),
        ],
        out_shape=[
            jax.ShapeDtypeStruct((B, 1, A), jnp.int32),
            jax.ShapeDtypeStruct((B, A, 4), jnp.float32),
            jax.ShapeDtypeStruct((B, 1, A), jnp.int32),
            jax.ShapeDtypeStruct((B, 1, A), jnp.float32),
        ],
        compiler_params=pltpu.CompilerParams(
            dimension_semantics=("arbitrary",)),
    )(anchors_t, gt_bboxes, lab, pd_t)

    tl = tl3[:, 0, :]
    fg = tl != _BG

    ts = _sc_scatter(si3.reshape(_B * _A), sv3.reshape(_B * _A))
    ts = ts.reshape(B, A, _NUM_CLASSES)
    return tl, tb, ts, fg
